# baseline (speedup 1.0000x reference)
import functools

import jax
import jax.numpy as jnp
from jax import lax
from jax.experimental import pallas as pl
from jax.experimental.pallas import tpu as pltpu
from jax.experimental.pallas import tpu_sc as plsc

NN = 10000
EE = 320000
D_IN = 128
D_H = 128
N_CLS = 64

NC = 2
NS = 16
NW = NC * NS
CHUNK = 128
NCHD = 80
NCHA = 160
EP = NW * NCHD * CHUNK
PAD_IDX = 10008
NP = 10240
ZR = NP // NS
NB = 4

_mesh = plsc.VectorSubcoreMesh(core_axis_name="c", subcore_axis_name="s")
_sc_params = pltpu.CompilerParams(use_tc_tiling_on_sc=False)


@functools.partial(
    pl.kernel,
    out_type=[
        jax.ShapeDtypeStruct((NC, NP, 16), jnp.float32),
        jax.ShapeDtypeStruct((NC, NP, 16), jnp.float32),
    ],
    mesh=_mesh,
    scratch_types=[
        pltpu.VMEM((NCHD, CHUNK), jnp.int32),
        pltpu.VMEM((NCHD, CHUNK), jnp.int32),
        pltpu.VMEM((CHUNK, 16), jnp.float32),
        pltpu.VMEM_SHARED((NP, 16), jnp.float32),
        pltpu.VMEM_SHARED((NP, 16), jnp.float32),
    ] + [pltpu.SemaphoreType.DMA] * NB,
    compiler_params=_sc_params,
)
def _deg_kernel(src_hbm, dst_hbm, ones_hbm, zeros_hbm, out_s, out_t,
                src_v, dst_v, ones_v, acc_s, acc_t, q0, q1, q2, q3):
    qs = [q0, q1, q2, q3]
    c = lax.axis_index("c")
    s = lax.axis_index("s")
    w = s * NC + c
    pltpu.sync_copy(zeros_hbm, acc_s.at[pl.ds(s * ZR, ZR)])
    pltpu.sync_copy(zeros_hbm, acc_t.at[pl.ds(s * ZR, ZR)])
    pltpu.sync_copy(src_hbm.at[w], src_v)
    pltpu.sync_copy(dst_hbm.at[w], dst_v)
    pltpu.sync_copy(ones_hbm, ones_v)
    plsc.subcore_barrier()

    def _pair_start(j, q):
        pltpu.async_copy(ones_v, acc_s.at[src_v.at[j]], qs[q], add=True)
        pltpu.async_copy(ones_v, acc_t.at[dst_v.at[j]], qs[q], add=True)

    def _pair_wait(j, q):
        pltpu.make_async_copy(ones_v, acc_s.at[src_v.at[j]], qs[q]).wait()
        pltpu.make_async_copy(ones_v, acc_t.at[dst_v.at[j]], qs[q]).wait()

    def body(r, carry):
        for b in range(NB):
            j = r * NB + b
            _pair_wait(j, b)
            _pair_start(j + NB, b)
        return carry

    for b in range(NB):
        _pair_start(b, b)
    lax.fori_loop(0, NCHD // NB - 1, body, 0)
    for b in range(NB):
        _pair_wait(NCHD - NB + b, b)
    plsc.subcore_barrier()
    pltpu.sync_copy(acc_s.at[pl.ds(s * ZR, ZR)], out_s.at[c, pl.ds(s * ZR, ZR)])
    pltpu.sync_copy(acc_t.at[pl.ds(s * ZR, ZR)], out_t.at[c, pl.ds(s * ZR, ZR)])


GS = 2


def _make_agg_kernel(D):
    HD = D // 2

    @functools.partial(
        pl.kernel,
        out_type=jax.ShapeDtypeStruct((NC, NP, HD), jnp.float32),
        mesh=_mesh,
        scratch_types=[
            pltpu.VMEM((NCHA, CHUNK), jnp.int32),
            pltpu.VMEM((NCHA, CHUNK), jnp.int32),
        ] + [pltpu.VMEM((CHUNK, HD), jnp.float32)] * (2 * GS) + [
            pltpu.VMEM_SHARED((NP, HD), jnp.float32),
        ] + [pltpu.SemaphoreType.DMA] * 4,
        compiler_params=_sc_params,
    )
    def _agg(h_hbm, src_hbm, dst_hbm, zeros_hbm, out_hbm,
             src_v, dst_v, *rest):
        bufa = list(rest[:GS])
        bufb = list(rest[GS:2 * GS])
        acc = rest[2 * GS]
        gA, gB, sA, sB = rest[2 * GS + 1:]
        c = lax.axis_index("c")
        s = lax.axis_index("s")
        pltpu.sync_copy(zeros_hbm, acc.at[pl.ds(s * ZR, ZR)])
        pltpu.sync_copy(src_hbm.at[c, s], src_v)
        pltpu.sync_copy(dst_hbm.at[s], dst_v)
        plsc.subcore_barrier()

        def _gstart(j, buf, sem):
            pltpu.async_copy(h_hbm.at[src_v.at[j]], buf, sem)

        def _gwait(j, buf, sem):
            pltpu.make_async_copy(h_hbm.at[src_v.at[j]], buf, sem).wait()

        def _sstart(j, buf, sem):
            pltpu.async_copy(buf, acc.at[dst_v.at[j]], sem, add=True)

        def _swait(j, buf, sem):
            pltpu.make_async_copy(buf, acc.at[dst_v.at[j]], sem).wait()

        W = 2 * GS
        R = NCHA // W

        for b in range(GS):
            _gstart(b, bufa[b], gA)
            _gstart(GS + b, bufb[b], gB)

        def _round(r, refill):
            for b in range(GS):
                _gwait(r * W + b, bufa[b], gA)
            for b in range(GS):
                _sstart(r * W + b, bufa[b], sA)
            for b in range(GS):
                _gwait(r * W + GS + b, bufb[b], gB)
            for b in range(GS):
                _sstart(r * W + GS + b, bufb[b], sB)
            for b in range(GS):
                _swait(r * W + b, bufa[b], sA)
            if refill:
                for b in range(GS):
                    _gstart((r + 1) * W + b, bufa[b], gA)
            for b in range(GS):
                _swait(r * W + GS + b, bufb[b], sB)
            if refill:
                for b in range(GS):
                    _gstart((r + 1) * W + GS + b, bufb[b], gB)

        def body(r, carry):
            _round(r, True)
            return carry

        lax.fori_loop(0, R - 1, body, 0)
        _round(R - 1, False)
        plsc.subcore_barrier()
        pltpu.sync_copy(acc.at[pl.ds(s * ZR, ZR)],
                        out_hbm.at[c, pl.ds(s * ZR, ZR)])

    return _agg


_BR = 256


def _norm_col(dpair, lo):
    d = dpair[0] + dpair[1]
    return lax.rsqrt(jnp.maximum(d[:, :1], lo))


def _tc1_body(x_ref, degs_ref, w1_ref, out_ref):
    ns = _norm_col(degs_ref[...], 1.0)
    out_ref[...] = jnp.dot(x_ref[...] * ns, w1_ref[...],
                           preferred_element_type=jnp.float32)


def _tc2_body(a_ref, degs_ref, degt_ref, w2_ref, b1_ref, out_ref):
    nd = _norm_col(degt_ref[...], 1.0)
    ns = _norm_col(degs_ref[...], 1.0)
    a = jnp.concatenate([a_ref[0], a_ref[1]], axis=1)
    h = jnp.maximum(a * nd + b1_ref[...], 0.0)
    out_ref[...] = jnp.dot(h * ns, w2_ref[...],
                           preferred_element_type=jnp.float32)


def _tc3_body(q_ref, degt_ref, b2_ref, out_ref):
    nd = _norm_col(degt_ref[...], 1.0)
    q = jnp.concatenate([q_ref[0], q_ref[1]], axis=1)
    out_ref[...] = q * nd + b2_ref[...]


def _row_spec(d):
    return pl.BlockSpec((_BR, d), lambda i: (i, 0))


def _deg_spec():
    return pl.BlockSpec((2, _BR, 16), lambda i: (0, i, 0))


def _full_spec(shape):
    return pl.BlockSpec(shape, lambda i: tuple(0 for _ in shape))


def kernel(x, edge_index, W1, b1, W2, b2):
    src = edge_index[0]
    dst = edge_index[1]
    pad = jnp.full((EP - EE,), PAD_IDX, jnp.int32)
    src_p = jnp.concatenate([src, pad])
    dst_p = jnp.concatenate([dst, pad])
    src_d = src_p.reshape(NW, NCHD, CHUNK)
    dst_d = dst_p.reshape(NW, NCHD, CHUNK)
    src2 = jnp.stack([2 * src_p, 2 * src_p + 1]).reshape(NC, NS, NCHA, CHUNK)
    dst_a = dst_p.reshape(NS, NCHA, CHUNK)
    x_p = jnp.zeros((NP, D_IN), jnp.float32).at[:NN].set(x)
    ones16 = jnp.ones((CHUNK, 16), jnp.float32)
    z16 = jnp.zeros((ZR, 16), jnp.float32)
    z64 = jnp.zeros((ZR, D_H // 2), jnp.float32)
    z32 = jnp.zeros((ZR, N_CLS // 2), jnp.float32)

    deg_s, deg_t = _deg_kernel(src_d, dst_d, ones16, z16)

    grid = (NP // _BR,)
    h1 = pl.pallas_call(
        _tc1_body,
        grid=grid,
        in_specs=[_row_spec(D_IN), _deg_spec(), _full_spec((D_IN, D_H))],
        out_specs=_row_spec(D_H),
        out_shape=jax.ShapeDtypeStruct((NP, D_H), jnp.float32),
    )(x_p, deg_s, W1)

    agg1 = _make_agg_kernel(D_H)(
        h1.reshape(2 * NP, D_H // 2), src2, dst_a, z64)

    g2 = pl.pallas_call(
        _tc2_body,
        grid=grid,
        in_specs=[pl.BlockSpec((2, _BR, D_H // 2), lambda i: (0, i, 0)),
                  _deg_spec(), _deg_spec(),
                  _full_spec((D_H, N_CLS)), _full_spec((1, D_H))],
        out_specs=_row_spec(N_CLS),
        out_shape=jax.ShapeDtypeStruct((NP, N_CLS), jnp.float32),
    )(agg1, deg_s, deg_t, W2, b1.reshape(1, D_H))

    agg2 = _make_agg_kernel(N_CLS)(
        g2.reshape(2 * NP, N_CLS // 2), src2, dst_a, z32)

    out = pl.pallas_call(
        _tc3_body,
        grid=grid,
        in_specs=[pl.BlockSpec((2, _BR, N_CLS // 2), lambda i: (0, i, 0)),
                  _deg_spec(), _full_spec((1, N_CLS))],
        out_specs=_row_spec(N_CLS),
        out_shape=jax.ShapeDtypeStruct((NP, N_CLS), jnp.float32),
    )(agg2, deg_t, b2.reshape(1, N_CLS))

    return out[:NN]


# speedup vs baseline: 1.0659x; 1.0659x over previous
import functools

import jax
import jax.numpy as jnp
from jax import lax
from jax.experimental import pallas as pl
from jax.experimental.pallas import tpu as pltpu
from jax.experimental.pallas import tpu_sc as plsc

NN = 10000
EE = 320000
D_IN = 128
D_H = 128
N_CLS = 64

NC = 2
NS = 16
NW = NC * NS
CHUNK = 128
NCHD = 80
NCHA = 160
EP = NW * NCHD * CHUNK
PAD_IDX = 10008
NP = 10240
ZR = NP // NS
NB = 4

_mesh = plsc.VectorSubcoreMesh(core_axis_name="c", subcore_axis_name="s")
_sc_params = pltpu.CompilerParams(use_tc_tiling_on_sc=False)


@functools.partial(
    pl.kernel,
    out_type=[
        jax.ShapeDtypeStruct((NC, NP, 16), jnp.float32),
        jax.ShapeDtypeStruct((NC, NP, 16), jnp.float32),
    ],
    mesh=_mesh,
    scratch_types=[
        pltpu.VMEM((NCHD, CHUNK), jnp.int32),
        pltpu.VMEM((NCHD, CHUNK), jnp.int32),
        pltpu.VMEM((CHUNK, 16), jnp.float32),
        pltpu.VMEM_SHARED((NP, 16), jnp.float32),
        pltpu.VMEM_SHARED((NP, 16), jnp.float32),
    ],
    compiler_params=_sc_params,
)
def _deg_kernel(src_hbm, dst_hbm, ones_hbm, zeros_hbm, out_s, out_t,
                src_v, dst_v, ones_v, acc_s, acc_t):
    c = lax.axis_index("c")
    s = lax.axis_index("s")
    w = s * NC + c
    pltpu.sync_copy(zeros_hbm, acc_s.at[pl.ds(s * ZR, ZR)])
    pltpu.sync_copy(zeros_hbm, acc_t.at[pl.ds(s * ZR, ZR)])
    pltpu.sync_copy(src_hbm.at[w], src_v)
    pltpu.sync_copy(dst_hbm.at[w], dst_v)
    pltpu.sync_copy(ones_hbm, ones_v)
    plsc.subcore_barrier()

    def body(j, carry):
        pltpu.sync_copy(ones_v, acc_s.at[src_v.at[j]], add=True)
        pltpu.sync_copy(ones_v, acc_t.at[dst_v.at[j]], add=True)
        return carry

    lax.fori_loop(0, NCHD, body, 0)
    plsc.subcore_barrier()
    pltpu.sync_copy(acc_s.at[pl.ds(s * ZR, ZR)], out_s.at[c, pl.ds(s * ZR, ZR)])
    pltpu.sync_copy(acc_t.at[pl.ds(s * ZR, ZR)], out_t.at[c, pl.ds(s * ZR, ZR)])


GS = 2


def _make_agg_kernel(D):
    HD = D // 2

    @functools.partial(
        pl.kernel,
        out_type=jax.ShapeDtypeStruct((NC, NP, HD), jnp.float32),
        mesh=_mesh,
        scratch_types=[
            pltpu.VMEM((NCHA, CHUNK), jnp.int32),
            pltpu.VMEM((NCHA, CHUNK), jnp.int32),
        ] + [pltpu.VMEM((CHUNK, HD), jnp.float32)] * (2 * GS) + [
            pltpu.VMEM_SHARED((NP, HD), jnp.float32),
        ] + [pltpu.SemaphoreType.DMA] * 4,
        compiler_params=_sc_params,
    )
    def _agg(h_hbm, src_hbm, dst_hbm, zeros_hbm, out_hbm,
             src_v, dst_v, *rest):
        NBUF = 2 * GS
        bufs = list(rest[:NBUF])
        acc = rest[NBUF]
        sems = list(rest[NBUF + 1:])
        c = lax.axis_index("c")
        s = lax.axis_index("s")
        pltpu.sync_copy(zeros_hbm, acc.at[pl.ds(s * ZR, ZR)])
        pltpu.sync_copy(src_hbm.at[c, s], src_v)
        pltpu.sync_copy(dst_hbm.at[s], dst_v)
        plsc.subcore_barrier()

        def _gstart(j, b):
            pltpu.async_copy(h_hbm.at[src_v.at[j]], bufs[b], sems[b])

        def _gwait(j, b):
            pltpu.make_async_copy(h_hbm.at[src_v.at[j]], bufs[b],
                                  sems[b]).wait()

        for b in range(NBUF):
            _gstart(b, b)

        R = NCHA // NBUF

        def body(r, carry):
            for b in range(NBUF):
                j = r * NBUF + b
                _gwait(j, b)
                pltpu.sync_copy(bufs[b], acc.at[dst_v.at[j]], add=True)
                @pl.when(r < R - 1)
                def _():
                    _gstart(j + NBUF, b)
            return carry

        lax.fori_loop(0, R, body, 0)
        plsc.subcore_barrier()
        pltpu.sync_copy(acc.at[pl.ds(s * ZR, ZR)],
                        out_hbm.at[c, pl.ds(s * ZR, ZR)])

    return _agg


_BR = 256


def _norm_col(dpair, lo):
    d = dpair[0] + dpair[1]
    return lax.rsqrt(jnp.maximum(d[:, :1], lo))


def _tc1_body(x_ref, degs_ref, w1_ref, out_ref):
    ns = _norm_col(degs_ref[...], 1.0)
    out_ref[...] = jnp.dot(x_ref[...] * ns, w1_ref[...],
                           preferred_element_type=jnp.float32)


def _tc2_body(a_ref, degs_ref, degt_ref, w2_ref, b1_ref, out_ref):
    nd = _norm_col(degt_ref[...], 1.0)
    ns = _norm_col(degs_ref[...], 1.0)
    a = jnp.concatenate([a_ref[0], a_ref[1]], axis=1)
    h = jnp.maximum(a * nd + b1_ref[...], 0.0)
    out_ref[...] = jnp.dot(h * ns, w2_ref[...],
                           preferred_element_type=jnp.float32)


def _tc3_body(q_ref, degt_ref, b2_ref, out_ref):
    nd = _norm_col(degt_ref[...], 1.0)
    q = jnp.concatenate([q_ref[0], q_ref[1]], axis=1)
    out_ref[...] = q * nd + b2_ref[...]


def _row_spec(d):
    return pl.BlockSpec((_BR, d), lambda i: (i, 0))


def _deg_spec():
    return pl.BlockSpec((2, _BR, 16), lambda i: (0, i, 0))


def _full_spec(shape):
    return pl.BlockSpec(shape, lambda i: tuple(0 for _ in shape))


def kernel(x, edge_index, W1, b1, W2, b2):
    src = edge_index[0]
    dst = edge_index[1]
    pad = jnp.full((EP - EE,), PAD_IDX, jnp.int32)
    src_p = jnp.concatenate([src, pad])
    dst_p = jnp.concatenate([dst, pad])
    src_d = src_p.reshape(NW, NCHD, CHUNK)
    dst_d = dst_p.reshape(NW, NCHD, CHUNK)
    src2 = jnp.stack([2 * src_p, 2 * src_p + 1]).reshape(NC, NS, NCHA, CHUNK)
    dst_a = dst_p.reshape(NS, NCHA, CHUNK)
    x_p = jnp.zeros((NP, D_IN), jnp.float32).at[:NN].set(x)
    ones16 = jnp.ones((CHUNK, 16), jnp.float32)
    z16 = jnp.zeros((ZR, 16), jnp.float32)
    z64 = jnp.zeros((ZR, D_H // 2), jnp.float32)
    z32 = jnp.zeros((ZR, N_CLS // 2), jnp.float32)

    deg_s, deg_t = _deg_kernel(src_d, dst_d, ones16, z16)

    grid = (NP // _BR,)
    h1 = pl.pallas_call(
        _tc1_body,
        grid=grid,
        in_specs=[_row_spec(D_IN), _deg_spec(), _full_spec((D_IN, D_H))],
        out_specs=_row_spec(D_H),
        out_shape=jax.ShapeDtypeStruct((NP, D_H), jnp.float32),
    )(x_p, deg_s, W1)

    agg1 = _make_agg_kernel(D_H)(
        h1.reshape(2 * NP, D_H // 2), src2, dst_a, z64)

    g2 = pl.pallas_call(
        _tc2_body,
        grid=grid,
        in_specs=[pl.BlockSpec((2, _BR, D_H // 2), lambda i: (0, i, 0)),
                  _deg_spec(), _deg_spec(),
                  _full_spec((D_H, N_CLS)), _full_spec((1, D_H))],
        out_specs=_row_spec(N_CLS),
        out_shape=jax.ShapeDtypeStruct((NP, N_CLS), jnp.float32),
    )(agg1, deg_s, deg_t, W2, b1.reshape(1, D_H))

    agg2 = _make_agg_kernel(N_CLS)(
        g2.reshape(2 * NP, N_CLS // 2), src2, dst_a, z32)

    out = pl.pallas_call(
        _tc3_body,
        grid=grid,
        in_specs=[pl.BlockSpec((2, _BR, N_CLS // 2), lambda i: (0, i, 0)),
                  _deg_spec(), _full_spec((1, N_CLS))],
        out_specs=_row_spec(N_CLS),
        out_shape=jax.ShapeDtypeStruct((NP, N_CLS), jnp.float32),
    )(agg2, deg_t, b2.reshape(1, N_CLS))

    return out[:NN]


# speedup vs baseline: 1.3712x; 1.2864x over previous
import functools

import jax
import jax.numpy as jnp
from jax import lax
from jax.experimental import pallas as pl
from jax.experimental.pallas import tpu as pltpu
from jax.experimental.pallas import tpu_sc as plsc

NN = 10000
EE = 320000
D_IN = 128
D_H = 128
N_CLS = 64

NC = 2
NS = 16
NW = NC * NS
CHUNK = 128
NCHD = 79
NCHA = 158
EP = NW * NCHD * CHUNK
PAD_IDX = 10008
NP = 10240
ZR = NP // NS
NB = 4

_mesh = plsc.VectorSubcoreMesh(core_axis_name="c", subcore_axis_name="s")
_sc_params = pltpu.CompilerParams(use_tc_tiling_on_sc=False)


@functools.partial(
    pl.kernel,
    out_type=[
        jax.ShapeDtypeStruct((NC, NP, 16), jnp.float32),
        jax.ShapeDtypeStruct((NC, NP, 16), jnp.float32),
    ],
    mesh=_mesh,
    scratch_types=[
        pltpu.VMEM((NCHD, CHUNK), jnp.int32),
        pltpu.VMEM((NCHD, CHUNK), jnp.int32),
        pltpu.VMEM((CHUNK, 16), jnp.float32),
        pltpu.VMEM_SHARED((NP, 16), jnp.float32),
        pltpu.VMEM_SHARED((NP, 16), jnp.float32),
    ],
    compiler_params=_sc_params,
)
def _deg_kernel(src_hbm, dst_hbm, ones_hbm, zeros_hbm, out_s, out_t,
                src_v, dst_v, ones_v, acc_s, acc_t):
    c = lax.axis_index("c")
    s = lax.axis_index("s")
    w = s * NC + c
    pltpu.sync_copy(zeros_hbm, acc_s.at[pl.ds(s * ZR, ZR)])
    pltpu.sync_copy(zeros_hbm, acc_t.at[pl.ds(s * ZR, ZR)])
    pltpu.sync_copy(src_hbm.at[w], src_v)
    pltpu.sync_copy(dst_hbm.at[w], dst_v)
    pltpu.sync_copy(ones_hbm, ones_v)
    plsc.subcore_barrier()

    def body(j, carry):
        pltpu.sync_copy(ones_v, acc_s.at[src_v.at[j]], add=True)
        pltpu.sync_copy(ones_v, acc_t.at[dst_v.at[j]], add=True)
        return carry

    lax.fori_loop(0, NCHD, body, 0)
    plsc.subcore_barrier()
    pltpu.sync_copy(acc_s.at[pl.ds(s * ZR, ZR)], out_s.at[c, pl.ds(s * ZR, ZR)])
    pltpu.sync_copy(acc_t.at[pl.ds(s * ZR, ZR)], out_t.at[c, pl.ds(s * ZR, ZR)])


def _make_agg_kernel(D):
    HD = D // 2

    @functools.partial(
        pl.kernel,
        out_type=jax.ShapeDtypeStruct((NC, NP, HD), jnp.float32),
        mesh=_mesh,
        scratch_types=[
            pltpu.VMEM((NCHA, CHUNK), jnp.int32),
            pltpu.VMEM((NCHA, CHUNK), jnp.int32),
            pltpu.VMEM((CHUNK, HD), jnp.float32),
            pltpu.VMEM((CHUNK, HD), jnp.float32),
            pltpu.VMEM_SHARED((NP, HD), jnp.float32),
            pltpu.SemaphoreType.DMA,
            pltpu.SemaphoreType.DMA,
        ],
        compiler_params=_sc_params,
    )
    def _agg(h_hbm, src_hbm, dst_hbm, zeros_hbm, out_hbm,
             src_v, dst_v, buf_a, buf_b, acc, sem_a, sem_b):
        c = lax.axis_index("c")
        s = lax.axis_index("s")
        pltpu.sync_copy(zeros_hbm, acc.at[pl.ds(s * ZR, ZR)])
        pltpu.sync_copy(src_hbm.at[c, s], src_v)
        pltpu.sync_copy(dst_hbm.at[s], dst_v)
        plsc.subcore_barrier()

        pltpu.async_copy(h_hbm.at[src_v.at[0]], buf_a, sem_a)

        def body(j, carry):
            @pl.when(j % 2 == 0)
            def _even():
                @pl.when(j + 1 < NCHA)
                def _():
                    pltpu.async_copy(h_hbm.at[src_v.at[j + 1]], buf_b, sem_b)
                pltpu.make_async_copy(h_hbm.at[src_v.at[j]], buf_a, sem_a).wait()
                pltpu.sync_copy(buf_a, acc.at[dst_v.at[j]], add=True)

            @pl.when(j % 2 == 1)
            def _odd():
                @pl.when(j + 1 < NCHA)
                def _():
                    pltpu.async_copy(h_hbm.at[src_v.at[j + 1]], buf_a, sem_a)
                pltpu.make_async_copy(h_hbm.at[src_v.at[j]], buf_b, sem_b).wait()
                pltpu.sync_copy(buf_b, acc.at[dst_v.at[j]], add=True)

            return carry

        lax.fori_loop(0, NCHA, body, 0)
        plsc.subcore_barrier()
        pltpu.sync_copy(acc.at[pl.ds(s * ZR, ZR)],
                        out_hbm.at[c, pl.ds(s * ZR, ZR)])

    return _agg


_BR = 256


def _norm_col(dpair, lo):
    d = dpair[0] + dpair[1]
    return lax.rsqrt(jnp.maximum(d[:, :1], lo))


def _tc1_body(x_ref, degs_ref, w1_ref, out_ref):
    ns = _norm_col(degs_ref[...], 1.0)
    out_ref[...] = jnp.dot(x_ref[...] * ns, w1_ref[...],
                           preferred_element_type=jnp.float32)


def _tc2_body(a_ref, degs_ref, degt_ref, w2_ref, b1_ref, out_ref):
    nd = _norm_col(degt_ref[...], 1.0)
    ns = _norm_col(degs_ref[...], 1.0)
    a = jnp.concatenate([a_ref[0], a_ref[1]], axis=1)
    h = jnp.maximum(a * nd + b1_ref[...], 0.0)
    out_ref[...] = jnp.dot(h * ns, w2_ref[...],
                           preferred_element_type=jnp.float32)


def _tc3_body(q_ref, degt_ref, b2_ref, out_ref):
    nd = _norm_col(degt_ref[...], 1.0)
    q = jnp.concatenate([q_ref[0], q_ref[1]], axis=1)
    out_ref[...] = q * nd + b2_ref[...]


def _row_spec(d):
    return pl.BlockSpec((_BR, d), lambda i: (i, 0))


def _deg_spec():
    return pl.BlockSpec((2, _BR, 16), lambda i: (0, i, 0))


def _full_spec(shape):
    return pl.BlockSpec(shape, lambda i: tuple(0 for _ in shape))


def kernel(x, edge_index, W1, b1, W2, b2):
    src = edge_index[0]
    dst = edge_index[1]
    pad = jnp.full((EP - EE,), PAD_IDX, jnp.int32)
    src_p = jnp.concatenate([src, pad])
    dst_p = jnp.concatenate([dst, pad])
    src_d = src_p.reshape(NW, NCHD, CHUNK)
    dst_d = dst_p.reshape(NW, NCHD, CHUNK)
    src2 = jnp.stack([2 * src_p, 2 * src_p + 1]).reshape(NC, NS, NCHA, CHUNK)
    dst_a = dst_p.reshape(NS, NCHA, CHUNK)
    x_p = jnp.zeros((NP, D_IN), jnp.float32).at[:NN].set(x)
    ones16 = jnp.ones((CHUNK, 16), jnp.float32)
    z16 = jnp.zeros((ZR, 16), jnp.float32)
    z64 = jnp.zeros((ZR, D_H // 2), jnp.float32)
    z32 = jnp.zeros((ZR, N_CLS // 2), jnp.float32)

    deg_s, deg_t = _deg_kernel(src_d, dst_d, ones16, z16)

    grid = (NP // _BR,)
    h1 = pl.pallas_call(
        _tc1_body,
        grid=grid,
        in_specs=[_row_spec(D_IN), _deg_spec(), _full_spec((D_IN, D_H))],
        out_specs=_row_spec(D_H),
        out_shape=jax.ShapeDtypeStruct((NP, D_H), jnp.float32),
    )(x_p, deg_s, W1)

    agg1 = _make_agg_kernel(D_H)(
        h1.reshape(2 * NP, D_H // 2), src2, dst_a, z64)

    g2 = pl.pallas_call(
        _tc2_body,
        grid=grid,
        in_specs=[pl.BlockSpec((2, _BR, D_H // 2), lambda i: (0, i, 0)),
                  _deg_spec(), _deg_spec(),
                  _full_spec((D_H, N_CLS)), _full_spec((1, D_H))],
        out_specs=_row_spec(N_CLS),
        out_shape=jax.ShapeDtypeStruct((NP, N_CLS), jnp.float32),
    )(agg1, deg_s, deg_t, W2, b1.reshape(1, D_H))

    agg2 = _make_agg_kernel(N_CLS)(
        g2.reshape(2 * NP, N_CLS // 2), src2, dst_a, z32)

    out = pl.pallas_call(
        _tc3_body,
        grid=grid,
        in_specs=[pl.BlockSpec((2, _BR, N_CLS // 2), lambda i: (0, i, 0)),
                  _deg_spec(), _full_spec((1, N_CLS))],
        out_specs=_row_spec(N_CLS),
        out_shape=jax.ShapeDtypeStruct((NP, N_CLS), jnp.float32),
    )(agg2, deg_t, b2.reshape(1, N_CLS))

    return out[:NN]


# speedup vs baseline: 1.4752x; 1.0758x over previous
import functools

import jax
import jax.numpy as jnp
from jax import lax
from jax.experimental import pallas as pl
from jax.experimental.pallas import tpu as pltpu
from jax.experimental.pallas import tpu_sc as plsc

NN = 10000
EE = 320000
D_IN = 128
D_H = 128
N_CLS = 64

NC = 2
NS = 16
NW = NC * NS
CHUNK = 128
NCHD = 79
NCHA = 158
EP = NW * NCHD * CHUNK
PAD_IDX = 10008
NP = 10240
ZR = NP // NS
NB = 4

_mesh = plsc.VectorSubcoreMesh(core_axis_name="c", subcore_axis_name="s")
_sc_params = pltpu.CompilerParams(use_tc_tiling_on_sc=False)


@functools.partial(
    pl.kernel,
    out_type=[
        jax.ShapeDtypeStruct((NC, NP, 16), jnp.float32),
        jax.ShapeDtypeStruct((NC, NP, 16), jnp.float32),
    ],
    mesh=_mesh,
    scratch_types=[
        pltpu.VMEM((NCHD, CHUNK), jnp.int32),
        pltpu.VMEM((NCHD, CHUNK), jnp.int32),
        pltpu.VMEM((CHUNK, 16), jnp.float32),
        pltpu.VMEM_SHARED((NP, 16), jnp.float32),
        pltpu.VMEM_SHARED((NP, 16), jnp.float32),
    ],
    compiler_params=_sc_params,
)
def _deg_kernel(src_hbm, dst_hbm, ones_hbm, zeros_hbm, out_s, out_t,
                src_v, dst_v, ones_v, acc_s, acc_t):
    c = lax.axis_index("c")
    s = lax.axis_index("s")
    w = s * NC + c
    pltpu.sync_copy(zeros_hbm, acc_s.at[pl.ds(s * ZR, ZR)])
    pltpu.sync_copy(zeros_hbm, acc_t.at[pl.ds(s * ZR, ZR)])
    pltpu.sync_copy(src_hbm.at[w], src_v)
    pltpu.sync_copy(dst_hbm.at[w], dst_v)
    pltpu.sync_copy(ones_hbm, ones_v)
    plsc.subcore_barrier()

    def body(j, carry):
        pltpu.sync_copy(ones_v, acc_s.at[src_v.at[j]], add=True)
        pltpu.sync_copy(ones_v, acc_t.at[dst_v.at[j]], add=True)
        return carry

    lax.fori_loop(0, NCHD, body, 0)
    plsc.subcore_barrier()
    pltpu.sync_copy(acc_s.at[pl.ds(s * ZR, ZR)], out_s.at[c, pl.ds(s * ZR, ZR)])
    pltpu.sync_copy(acc_t.at[pl.ds(s * ZR, ZR)], out_t.at[c, pl.ds(s * ZR, ZR)])


def _make_agg_kernel(D):
    HD = D // 2

    @functools.partial(
        pl.kernel,
        out_type=jax.ShapeDtypeStruct((NC, NP, HD), jnp.float32),
        mesh=_mesh,
        scratch_types=[
            pltpu.VMEM((NCHA, CHUNK), jnp.int32),
            pltpu.VMEM((NCHA, CHUNK), jnp.int32),
            pltpu.VMEM((CHUNK, HD), jnp.float32),
            pltpu.VMEM((CHUNK, HD), jnp.float32),
            pltpu.VMEM((CHUNK, HD), jnp.float32),
            pltpu.VMEM_SHARED((NP, HD), jnp.float32),
            pltpu.SemaphoreType.DMA,
            pltpu.SemaphoreType.DMA,
            pltpu.SemaphoreType.DMA,
        ],
        compiler_params=_sc_params,
    )
    def _agg(h_hbm, src_hbm, dst_hbm, zeros_hbm, out_hbm,
             src_v, dst_v, buf_a, buf_b, buf_c, acc, sem_a, sem_b, sem_c):
        bufs = [buf_a, buf_b, buf_c]
        sems = [sem_a, sem_b, sem_c]
        c = lax.axis_index("c")
        s = lax.axis_index("s")
        pltpu.sync_copy(zeros_hbm, acc.at[pl.ds(s * ZR, ZR)])
        pltpu.sync_copy(src_hbm.at[c, s], src_v)
        pltpu.sync_copy(dst_hbm.at[s], dst_v)
        plsc.subcore_barrier()

        pltpu.async_copy(h_hbm.at[src_v.at[0]], bufs[0], sems[0])
        pltpu.async_copy(h_hbm.at[src_v.at[1]], bufs[1], sems[1])

        def body(j, carry):
            for p in range(3):
                @pl.when(j % 3 == p)
                def _(p=p):
                    @pl.when(j + 2 < NCHA)
                    def _():
                        pltpu.async_copy(h_hbm.at[src_v.at[j + 2]],
                                         bufs[(p + 2) % 3], sems[(p + 2) % 3])
                    pltpu.make_async_copy(h_hbm.at[src_v.at[j]],
                                          bufs[p], sems[p]).wait()
                    pltpu.sync_copy(bufs[p], acc.at[dst_v.at[j]], add=True)
            return carry

        lax.fori_loop(0, NCHA, body, 0)
        plsc.subcore_barrier()
        pltpu.sync_copy(acc.at[pl.ds(s * ZR, ZR)],
                        out_hbm.at[c, pl.ds(s * ZR, ZR)])

    return _agg


_BR = 256


def _norm_col(dpair, lo):
    d = dpair[0] + dpair[1]
    return lax.rsqrt(jnp.maximum(d[:, :1], lo))


def _tc1_body(x_ref, degs_ref, w1_ref, out_ref):
    ns = _norm_col(degs_ref[...], 1.0)
    out_ref[...] = jnp.dot(x_ref[...] * ns, w1_ref[...],
                           preferred_element_type=jnp.float32)


def _tc2_body(a_ref, degs_ref, degt_ref, w2_ref, b1_ref, out_ref):
    nd = _norm_col(degt_ref[...], 1.0)
    ns = _norm_col(degs_ref[...], 1.0)
    a = jnp.concatenate([a_ref[0], a_ref[1]], axis=1)
    h = jnp.maximum(a * nd + b1_ref[...], 0.0)
    out_ref[...] = jnp.dot(h * ns, w2_ref[...],
                           preferred_element_type=jnp.float32)


def _tc3_body(q_ref, degt_ref, b2_ref, out_ref):
    nd = _norm_col(degt_ref[...], 1.0)
    q = jnp.concatenate([q_ref[0], q_ref[1]], axis=1)
    out_ref[...] = q * nd + b2_ref[...]


def _row_spec(d):
    return pl.BlockSpec((_BR, d), lambda i: (i, 0))


def _deg_spec():
    return pl.BlockSpec((2, _BR, 16), lambda i: (0, i, 0))


def _full_spec(shape):
    return pl.BlockSpec(shape, lambda i: tuple(0 for _ in shape))


def kernel(x, edge_index, W1, b1, W2, b2):
    src = edge_index[0]
    dst = edge_index[1]
    pad = jnp.full((EP - EE,), PAD_IDX, jnp.int32)
    src_p = jnp.concatenate([src, pad])
    dst_p = jnp.concatenate([dst, pad])
    src_d = src_p.reshape(NW, NCHD, CHUNK)
    dst_d = dst_p.reshape(NW, NCHD, CHUNK)
    src2 = jnp.stack([2 * src_p, 2 * src_p + 1]).reshape(NC, NS, NCHA, CHUNK)
    dst_a = dst_p.reshape(NS, NCHA, CHUNK)
    x_p = jnp.zeros((NP, D_IN), jnp.float32).at[:NN].set(x)
    ones16 = jnp.ones((CHUNK, 16), jnp.float32)
    z16 = jnp.zeros((ZR, 16), jnp.float32)
    z64 = jnp.zeros((ZR, D_H // 2), jnp.float32)
    z32 = jnp.zeros((ZR, N_CLS // 2), jnp.float32)

    deg_s, deg_t = _deg_kernel(src_d, dst_d, ones16, z16)

    grid = (NP // _BR,)
    h1 = pl.pallas_call(
        _tc1_body,
        grid=grid,
        in_specs=[_row_spec(D_IN), _deg_spec(), _full_spec((D_IN, D_H))],
        out_specs=_row_spec(D_H),
        out_shape=jax.ShapeDtypeStruct((NP, D_H), jnp.float32),
    )(x_p, deg_s, W1)

    agg1 = _make_agg_kernel(D_H)(
        h1.reshape(2 * NP, D_H // 2), src2, dst_a, z64)

    g2 = pl.pallas_call(
        _tc2_body,
        grid=grid,
        in_specs=[pl.BlockSpec((2, _BR, D_H // 2), lambda i: (0, i, 0)),
                  _deg_spec(), _deg_spec(),
                  _full_spec((D_H, N_CLS)), _full_spec((1, D_H))],
        out_specs=_row_spec(N_CLS),
        out_shape=jax.ShapeDtypeStruct((NP, N_CLS), jnp.float32),
    )(agg1, deg_s, deg_t, W2, b1.reshape(1, D_H))

    agg2 = _make_agg_kernel(N_CLS)(
        g2.reshape(2 * NP, N_CLS // 2), src2, dst_a, z32)

    out = pl.pallas_call(
        _tc3_body,
        grid=grid,
        in_specs=[pl.BlockSpec((2, _BR, N_CLS // 2), lambda i: (0, i, 0)),
                  _deg_spec(), _full_spec((1, N_CLS))],
        out_specs=_row_spec(N_CLS),
        out_shape=jax.ShapeDtypeStruct((NP, N_CLS), jnp.float32),
    )(agg2, deg_t, b2.reshape(1, N_CLS))

    return out[:NN]


# speedup vs baseline: 1.5077x; 1.0221x over previous
import functools

import jax
import jax.numpy as jnp
from jax import lax
from jax.experimental import pallas as pl
from jax.experimental.pallas import tpu as pltpu
from jax.experimental.pallas import tpu_sc as plsc

NN = 10000
EE = 320000
D_IN = 128
D_H = 128
N_CLS = 64

NC = 2
NS = 16
NW = NC * NS
CHUNK = 128
NCHD = 79
NCHA = 158
EP = NW * NCHD * CHUNK
PAD_IDX = 10008
NP = 10240
ZR = NP // NS
NB = 4

_mesh = plsc.VectorSubcoreMesh(core_axis_name="c", subcore_axis_name="s")
_sc_params = pltpu.CompilerParams(use_tc_tiling_on_sc=False)


@functools.partial(
    pl.kernel,
    out_type=[
        jax.ShapeDtypeStruct((NC, NP, 16), jnp.float32),
        jax.ShapeDtypeStruct((NC, NP, 16), jnp.float32),
    ],
    mesh=_mesh,
    scratch_types=[
        pltpu.VMEM((NCHD, CHUNK), jnp.int32),
        pltpu.VMEM((NCHD, CHUNK), jnp.int32),
        pltpu.VMEM((CHUNK, 16), jnp.float32),
        pltpu.VMEM_SHARED((NP, 16), jnp.float32),
        pltpu.VMEM_SHARED((NP, 16), jnp.float32),
    ],
    compiler_params=_sc_params,
)
def _deg_kernel(src_hbm, dst_hbm, ones_hbm, zeros_hbm, out_s, out_t,
                src_v, dst_v, ones_v, acc_s, acc_t):
    c = lax.axis_index("c")
    s = lax.axis_index("s")
    w = s * NC + c
    pltpu.sync_copy(zeros_hbm, acc_s.at[pl.ds(s * ZR, ZR)])
    pltpu.sync_copy(zeros_hbm, acc_t.at[pl.ds(s * ZR, ZR)])
    pltpu.sync_copy(src_hbm.at[w], src_v)
    pltpu.sync_copy(dst_hbm.at[w], dst_v)
    pltpu.sync_copy(ones_hbm, ones_v)
    plsc.subcore_barrier()

    def body(j, carry):
        pltpu.sync_copy(ones_v, acc_s.at[src_v.at[j]], add=True)
        pltpu.sync_copy(ones_v, acc_t.at[dst_v.at[j]], add=True)
        return carry

    lax.fori_loop(0, NCHD, body, 0)
    plsc.subcore_barrier()
    pltpu.sync_copy(acc_s.at[pl.ds(s * ZR, ZR)], out_s.at[c, pl.ds(s * ZR, ZR)])
    pltpu.sync_copy(acc_t.at[pl.ds(s * ZR, ZR)], out_t.at[c, pl.ds(s * ZR, ZR)])


PF = 4


def _make_agg_kernel(D):
    HD = D // 2

    @functools.partial(
        pl.kernel,
        out_type=jax.ShapeDtypeStruct((NC, NP, HD), jnp.float32),
        mesh=_mesh,
        scratch_types=[
            pltpu.VMEM((NCHA, CHUNK), jnp.int32),
            pltpu.VMEM((NCHA, CHUNK), jnp.int32),
        ] + [pltpu.VMEM((CHUNK, HD), jnp.float32)] * PF + [
            pltpu.VMEM_SHARED((NP, HD), jnp.float32),
        ] + [pltpu.SemaphoreType.DMA] * PF,
        compiler_params=_sc_params,
    )
    def _agg(h_hbm, src_hbm, dst_hbm, zeros_hbm, out_hbm,
             src_v, dst_v, *rest):
        bufs = list(rest[:PF])
        acc = rest[PF]
        sems = list(rest[PF + 1:])
        c = lax.axis_index("c")
        s = lax.axis_index("s")
        pltpu.sync_copy(zeros_hbm, acc.at[pl.ds(s * ZR, ZR)])
        pltpu.sync_copy(src_hbm.at[c, s], src_v)
        pltpu.sync_copy(dst_hbm.at[s], dst_v)
        plsc.subcore_barrier()

        for p in range(PF - 1):
            pltpu.async_copy(h_hbm.at[src_v.at[p]], bufs[p], sems[p])

        def body(j, carry):
            for p in range(PF):
                @pl.when(j % PF == p)
                def _(p=p):
                    @pl.when(j + PF - 1 < NCHA)
                    def _():
                        pltpu.async_copy(h_hbm.at[src_v.at[j + PF - 1]],
                                         bufs[(p + PF - 1) % PF],
                                         sems[(p + PF - 1) % PF])
                    pltpu.make_async_copy(h_hbm.at[src_v.at[j]],
                                          bufs[p], sems[p]).wait()
                    pltpu.sync_copy(bufs[p], acc.at[dst_v.at[j]], add=True)
            return carry

        lax.fori_loop(0, NCHA, body, 0)
        plsc.subcore_barrier()
        pltpu.sync_copy(acc.at[pl.ds(s * ZR, ZR)],
                        out_hbm.at[c, pl.ds(s * ZR, ZR)])

    return _agg


_BR = 256


def _norm_col(dpair, lo):
    d = dpair[0] + dpair[1]
    return lax.rsqrt(jnp.maximum(d[:, :1], lo))


def _tc1_body(x_ref, degs_ref, w1_ref, out_ref):
    ns = _norm_col(degs_ref[...], 1.0)
    out_ref[...] = jnp.dot(x_ref[...] * ns, w1_ref[...],
                           preferred_element_type=jnp.float32)


def _tc2_body(a_ref, degs_ref, degt_ref, w2_ref, b1_ref, out_ref):
    nd = _norm_col(degt_ref[...], 1.0)
    ns = _norm_col(degs_ref[...], 1.0)
    a = jnp.concatenate([a_ref[0], a_ref[1]], axis=1)
    h = jnp.maximum(a * nd + b1_ref[...], 0.0)
    out_ref[...] = jnp.dot(h * ns, w2_ref[...],
                           preferred_element_type=jnp.float32)


def _tc3_body(q_ref, degt_ref, b2_ref, out_ref):
    nd = _norm_col(degt_ref[...], 1.0)
    q = jnp.concatenate([q_ref[0], q_ref[1]], axis=1)
    out_ref[...] = q * nd + b2_ref[...]


def _row_spec(d):
    return pl.BlockSpec((_BR, d), lambda i: (i, 0))


def _deg_spec():
    return pl.BlockSpec((2, _BR, 16), lambda i: (0, i, 0))


def _full_spec(shape):
    return pl.BlockSpec(shape, lambda i: tuple(0 for _ in shape))


def kernel(x, edge_index, W1, b1, W2, b2):
    src = edge_index[0]
    dst = edge_index[1]
    pad = jnp.full((EP - EE,), PAD_IDX, jnp.int32)
    src_p = jnp.concatenate([src, pad])
    dst_p = jnp.concatenate([dst, pad])
    src_d = src_p.reshape(NW, NCHD, CHUNK)
    dst_d = dst_p.reshape(NW, NCHD, CHUNK)
    src2 = jnp.stack([2 * src_p, 2 * src_p + 1]).reshape(NC, NS, NCHA, CHUNK)
    dst_a = dst_p.reshape(NS, NCHA, CHUNK)
    x_p = jnp.zeros((NP, D_IN), jnp.float32).at[:NN].set(x)
    ones16 = jnp.ones((CHUNK, 16), jnp.float32)
    z16 = jnp.zeros((ZR, 16), jnp.float32)
    z64 = jnp.zeros((ZR, D_H // 2), jnp.float32)
    z32 = jnp.zeros((ZR, N_CLS // 2), jnp.float32)

    deg_s, deg_t = _deg_kernel(src_d, dst_d, ones16, z16)

    grid = (NP // _BR,)
    h1 = pl.pallas_call(
        _tc1_body,
        grid=grid,
        in_specs=[_row_spec(D_IN), _deg_spec(), _full_spec((D_IN, D_H))],
        out_specs=_row_spec(D_H),
        out_shape=jax.ShapeDtypeStruct((NP, D_H), jnp.float32),
    )(x_p, deg_s, W1)

    agg1 = _make_agg_kernel(D_H)(
        h1.reshape(2 * NP, D_H // 2), src2, dst_a, z64)

    g2 = pl.pallas_call(
        _tc2_body,
        grid=grid,
        in_specs=[pl.BlockSpec((2, _BR, D_H // 2), lambda i: (0, i, 0)),
                  _deg_spec(), _deg_spec(),
                  _full_spec((D_H, N_CLS)), _full_spec((1, D_H))],
        out_specs=_row_spec(N_CLS),
        out_shape=jax.ShapeDtypeStruct((NP, N_CLS), jnp.float32),
    )(agg1, deg_s, deg_t, W2, b1.reshape(1, D_H))

    agg2 = _make_agg_kernel(N_CLS)(
        g2.reshape(2 * NP, N_CLS // 2), src2, dst_a, z32)

    out = pl.pallas_call(
        _tc3_body,
        grid=grid,
        in_specs=[pl.BlockSpec((2, _BR, N_CLS // 2), lambda i: (0, i, 0)),
                  _deg_spec(), _full_spec((1, N_CLS))],
        out_specs=_row_spec(N_CLS),
        out_shape=jax.ShapeDtypeStruct((NP, N_CLS), jnp.float32),
    )(agg2, deg_t, b2.reshape(1, N_CLS))

    return out[:NN]


# speedup vs baseline: 1.5086x; 1.0006x over previous
import functools

import jax
import jax.numpy as jnp
from jax import lax
from jax.experimental import pallas as pl
from jax.experimental.pallas import tpu as pltpu
from jax.experimental.pallas import tpu_sc as plsc

NN = 10000
EE = 320000
D_IN = 128
D_H = 128
N_CLS = 64

NC = 2
NS = 16
NW = NC * NS
CHUNK = 128
NCHD = 79
NCHA = 158
EP = NW * NCHD * CHUNK
PAD_IDX = 10008
NP = 10240
ZR = NP // NS
NB = 4

_mesh = plsc.VectorSubcoreMesh(core_axis_name="c", subcore_axis_name="s")
_sc_params = pltpu.CompilerParams(use_tc_tiling_on_sc=False)


@functools.partial(
    pl.kernel,
    out_type=[
        jax.ShapeDtypeStruct((NC, NP, 16), jnp.float32),
        jax.ShapeDtypeStruct((NC, NP, 16), jnp.float32),
    ],
    mesh=_mesh,
    scratch_types=[
        pltpu.VMEM((NCHD, CHUNK), jnp.int32),
        pltpu.VMEM((NCHD, CHUNK), jnp.int32),
        pltpu.VMEM((CHUNK, 16), jnp.float32),
        pltpu.VMEM_SHARED((NP, 16), jnp.float32),
        pltpu.VMEM_SHARED((NP, 16), jnp.float32),
    ],
    compiler_params=_sc_params,
)
def _deg_kernel(src_hbm, dst_hbm, ones_hbm, zeros_hbm, out_s, out_t,
                src_v, dst_v, ones_v, acc_s, acc_t):
    c = lax.axis_index("c")
    s = lax.axis_index("s")
    w = s * NC + c
    pltpu.sync_copy(zeros_hbm, acc_s.at[pl.ds(s * ZR, ZR)])
    pltpu.sync_copy(zeros_hbm, acc_t.at[pl.ds(s * ZR, ZR)])
    pltpu.sync_copy(src_hbm.at[w], src_v)
    pltpu.sync_copy(dst_hbm.at[w], dst_v)
    pltpu.sync_copy(ones_hbm, ones_v)
    plsc.subcore_barrier()

    def body(j, carry):
        pltpu.sync_copy(ones_v, acc_s.at[src_v.at[j]], add=True)
        pltpu.sync_copy(ones_v, acc_t.at[dst_v.at[j]], add=True)
        return carry

    lax.fori_loop(0, NCHD, body, 0)
    plsc.subcore_barrier()
    pltpu.sync_copy(acc_s.at[pl.ds(s * ZR, ZR)], out_s.at[c, pl.ds(s * ZR, ZR)])
    pltpu.sync_copy(acc_t.at[pl.ds(s * ZR, ZR)], out_t.at[c, pl.ds(s * ZR, ZR)])


PF = 4


def _make_agg_kernel(D):
    HD = D // 2

    @functools.partial(
        pl.kernel,
        out_type=jax.ShapeDtypeStruct((NC, NP, HD), jnp.float32),
        mesh=_mesh,
        scratch_types=[
            pltpu.VMEM((NCHA, CHUNK), jnp.int32),
            pltpu.VMEM((NCHA, CHUNK), jnp.int32),
        ] + [pltpu.VMEM((CHUNK, HD), jnp.float32)] * PF + [
            pltpu.VMEM_SHARED((NP, HD), jnp.float32),
        ] + [pltpu.SemaphoreType.DMA] * (2 * PF),
        compiler_params=_sc_params,
    )
    def _agg(h_hbm, src_hbm, dst_hbm, zeros_hbm, out_hbm,
             src_v, dst_v, *rest):
        bufs = list(rest[:PF])
        acc = rest[PF]
        sems = list(rest[PF + 1:PF + 1 + PF])
        ssems = list(rest[PF + 1 + PF:])
        c = lax.axis_index("c")
        s = lax.axis_index("s")
        pltpu.sync_copy(zeros_hbm, acc.at[pl.ds(s * ZR, ZR)])
        pltpu.sync_copy(src_hbm.at[c, s], src_v)
        pltpu.sync_copy(dst_hbm.at[s], dst_v)
        plsc.subcore_barrier()

        for p in range(PF - 1):
            pltpu.async_copy(h_hbm.at[src_v.at[p]], bufs[p], sems[p])

        def body(j, carry):
            for p in range(PF):
                @pl.when(j % PF == p)
                def _(p=p):
                    q = (p + PF - 1) % PF

                    @pl.when(j > 0)
                    def _():
                        pltpu.make_async_copy(
                            bufs[q], acc.at[dst_v.at[j - 1]], ssems[q]).wait()

                    @pl.when(j + PF - 1 < NCHA)
                    def _():
                        pltpu.async_copy(h_hbm.at[src_v.at[j + PF - 1]],
                                         bufs[q], sems[q])
                    pltpu.make_async_copy(h_hbm.at[src_v.at[j]],
                                          bufs[p], sems[p]).wait()
                    pltpu.async_copy(bufs[p], acc.at[dst_v.at[j]],
                                     ssems[p], add=True)
            return carry

        lax.fori_loop(0, NCHA, body, 0)
        pltpu.make_async_copy(bufs[(NCHA - 1) % PF],
                              acc.at[dst_v.at[NCHA - 1]],
                              ssems[(NCHA - 1) % PF]).wait()
        plsc.subcore_barrier()
        pltpu.sync_copy(acc.at[pl.ds(s * ZR, ZR)],
                        out_hbm.at[c, pl.ds(s * ZR, ZR)])

    return _agg


_BR = 256


def _norm_col(dpair, lo):
    d = dpair[0] + dpair[1]
    return lax.rsqrt(jnp.maximum(d[:, :1], lo))


def _tc1_body(x_ref, degs_ref, w1_ref, out_ref):
    ns = _norm_col(degs_ref[...], 1.0)
    out_ref[...] = jnp.dot(x_ref[...] * ns, w1_ref[...],
                           preferred_element_type=jnp.float32)


def _tc2_body(a_ref, degs_ref, degt_ref, w2_ref, b1_ref, out_ref):
    nd = _norm_col(degt_ref[...], 1.0)
    ns = _norm_col(degs_ref[...], 1.0)
    a = jnp.concatenate([a_ref[0], a_ref[1]], axis=1)
    h = jnp.maximum(a * nd + b1_ref[...], 0.0)
    out_ref[...] = jnp.dot(h * ns, w2_ref[...],
                           preferred_element_type=jnp.float32)


def _tc3_body(q_ref, degt_ref, b2_ref, out_ref):
    nd = _norm_col(degt_ref[...], 1.0)
    q = jnp.concatenate([q_ref[0], q_ref[1]], axis=1)
    out_ref[...] = q * nd + b2_ref[...]


def _row_spec(d):
    return pl.BlockSpec((_BR, d), lambda i: (i, 0))


def _deg_spec():
    return pl.BlockSpec((2, _BR, 16), lambda i: (0, i, 0))


def _full_spec(shape):
    return pl.BlockSpec(shape, lambda i: tuple(0 for _ in shape))


def kernel(x, edge_index, W1, b1, W2, b2):
    src = edge_index[0]
    dst = edge_index[1]
    pad = jnp.full((EP - EE,), PAD_IDX, jnp.int32)
    src_p = jnp.concatenate([src, pad])
    dst_p = jnp.concatenate([dst, pad])
    src_d = src_p.reshape(NW, NCHD, CHUNK)
    dst_d = dst_p.reshape(NW, NCHD, CHUNK)
    src2 = jnp.stack([2 * src_p, 2 * src_p + 1]).reshape(NC, NS, NCHA, CHUNK)
    dst_a = dst_p.reshape(NS, NCHA, CHUNK)
    x_p = jnp.zeros((NP, D_IN), jnp.float32).at[:NN].set(x)
    ones16 = jnp.ones((CHUNK, 16), jnp.float32)
    z16 = jnp.zeros((ZR, 16), jnp.float32)
    z64 = jnp.zeros((ZR, D_H // 2), jnp.float32)
    z32 = jnp.zeros((ZR, N_CLS // 2), jnp.float32)

    deg_s, deg_t = _deg_kernel(src_d, dst_d, ones16, z16)

    grid = (NP // _BR,)
    h1 = pl.pallas_call(
        _tc1_body,
        grid=grid,
        in_specs=[_row_spec(D_IN), _deg_spec(), _full_spec((D_IN, D_H))],
        out_specs=_row_spec(D_H),
        out_shape=jax.ShapeDtypeStruct((NP, D_H), jnp.float32),
    )(x_p, deg_s, W1)

    agg1 = _make_agg_kernel(D_H)(
        h1.reshape(2 * NP, D_H // 2), src2, dst_a, z64)

    g2 = pl.pallas_call(
        _tc2_body,
        grid=grid,
        in_specs=[pl.BlockSpec((2, _BR, D_H // 2), lambda i: (0, i, 0)),
                  _deg_spec(), _deg_spec(),
                  _full_spec((D_H, N_CLS)), _full_spec((1, D_H))],
        out_specs=_row_spec(N_CLS),
        out_shape=jax.ShapeDtypeStruct((NP, N_CLS), jnp.float32),
    )(agg1, deg_s, deg_t, W2, b1.reshape(1, D_H))

    agg2 = _make_agg_kernel(N_CLS)(
        g2.reshape(2 * NP, N_CLS // 2), src2, dst_a, z32)

    out = pl.pallas_call(
        _tc3_body,
        grid=grid,
        in_specs=[pl.BlockSpec((2, _BR, N_CLS // 2), lambda i: (0, i, 0)),
                  _deg_spec(), _full_spec((1, N_CLS))],
        out_specs=_row_spec(N_CLS),
        out_shape=jax.ShapeDtypeStruct((NP, N_CLS), jnp.float32),
    )(agg2, deg_t, b2.reshape(1, N_CLS))

    return out[:NN]


# speedup vs baseline: 1.5481x; 1.0262x over previous
import functools

import jax
import jax.numpy as jnp
from jax import lax
from jax.experimental import pallas as pl
from jax.experimental.pallas import tpu as pltpu
from jax.experimental.pallas import tpu_sc as plsc

NN = 10000
EE = 320000
D_IN = 128
D_H = 128
N_CLS = 64

NC = 2
NS = 16
NW = NC * NS
CHUNK = 128
NCHD = 79
NCHA = 158
EP = NW * NCHD * CHUNK
PAD_IDX = 10008
NP = 10240
ZR = NP // NS
NB = 4

_mesh = plsc.VectorSubcoreMesh(core_axis_name="c", subcore_axis_name="s")
_sc_params = pltpu.CompilerParams(use_tc_tiling_on_sc=False)


@functools.partial(
    pl.kernel,
    out_type=[
        jax.ShapeDtypeStruct((NC, NP, 16), jnp.float32),
        jax.ShapeDtypeStruct((NC, NP, 16), jnp.float32),
    ],
    mesh=_mesh,
    scratch_types=[
        pltpu.VMEM((NCHD, CHUNK), jnp.int32),
        pltpu.VMEM((NCHD, CHUNK), jnp.int32),
        pltpu.VMEM((CHUNK, 16), jnp.float32),
        pltpu.VMEM_SHARED((NP, 16), jnp.float32),
        pltpu.VMEM_SHARED((NP, 16), jnp.float32),
        pltpu.SemaphoreType.DMA,
        pltpu.SemaphoreType.DMA,
    ],
    compiler_params=_sc_params,
)
def _deg_kernel(src_hbm, dst_hbm, ones_hbm, zeros_hbm, out_s, out_t,
                src_v, dst_v, ones_v, acc_s, acc_t, q0, q1):
    qs = [q0, q1]
    c = lax.axis_index("c")
    s = lax.axis_index("s")
    w = s * NC + c
    pltpu.sync_copy(zeros_hbm, acc_s.at[pl.ds(s * ZR, ZR)])
    pltpu.sync_copy(zeros_hbm, acc_t.at[pl.ds(s * ZR, ZR)])
    pltpu.sync_copy(src_hbm.at[w], src_v)
    pltpu.sync_copy(dst_hbm.at[w], dst_v)
    pltpu.sync_copy(ones_hbm, ones_v)
    plsc.subcore_barrier()

    def body(j, carry):
        for p in range(2):
            @pl.when(j % 2 == p)
            def _(p=p):
                @pl.when(j >= 2)
                def _():
                    pltpu.make_async_copy(
                        ones_v, acc_s.at[src_v.at[j - 2]], qs[p]).wait()
                    pltpu.make_async_copy(
                        ones_v, acc_t.at[dst_v.at[j - 2]], qs[p]).wait()
                pltpu.async_copy(ones_v, acc_s.at[src_v.at[j]], qs[p],
                                 add=True)
                pltpu.async_copy(ones_v, acc_t.at[dst_v.at[j]], qs[p],
                                 add=True)
        return carry

    lax.fori_loop(0, NCHD, body, 0)
    for j in (NCHD - 2, NCHD - 1):
        pltpu.make_async_copy(ones_v, acc_s.at[src_v.at[j]], qs[j % 2]).wait()
        pltpu.make_async_copy(ones_v, acc_t.at[dst_v.at[j]], qs[j % 2]).wait()
    plsc.subcore_barrier()
    pltpu.sync_copy(acc_s.at[pl.ds(s * ZR, ZR)], out_s.at[c, pl.ds(s * ZR, ZR)])
    pltpu.sync_copy(acc_t.at[pl.ds(s * ZR, ZR)], out_t.at[c, pl.ds(s * ZR, ZR)])


PF = 5


def _make_agg_kernel(D):
    HD = D // 2

    @functools.partial(
        pl.kernel,
        out_type=jax.ShapeDtypeStruct((NC, NP, HD), jnp.float32),
        mesh=_mesh,
        scratch_types=[
            pltpu.VMEM((NCHA, CHUNK), jnp.int32),
            pltpu.VMEM((NCHA, CHUNK), jnp.int32),
        ] + [pltpu.VMEM((CHUNK, HD), jnp.float32)] * PF + [
            pltpu.VMEM_SHARED((NP, HD), jnp.float32),
        ] + [pltpu.SemaphoreType.DMA] * (2 * PF),
        compiler_params=_sc_params,
    )
    def _agg(h_hbm, src_hbm, dst_hbm, zeros_hbm, out_hbm,
             src_v, dst_v, *rest):
        bufs = list(rest[:PF])
        acc = rest[PF]
        sems = list(rest[PF + 1:PF + 1 + PF])
        ssems = list(rest[PF + 1 + PF:])
        c = lax.axis_index("c")
        s = lax.axis_index("s")
        pltpu.sync_copy(zeros_hbm, acc.at[pl.ds(s * ZR, ZR)])
        pltpu.sync_copy(src_hbm.at[c, s], src_v)
        pltpu.sync_copy(dst_hbm.at[s], dst_v)
        plsc.subcore_barrier()

        for p in range(PF - 1):
            pltpu.async_copy(h_hbm.at[src_v.at[p]], bufs[p], sems[p])

        def body(j, carry):
            for p in range(PF):
                @pl.when(j % PF == p)
                def _(p=p):
                    q = (p + PF - 1) % PF

                    @pl.when(j > 0)
                    def _():
                        pltpu.make_async_copy(
                            bufs[q], acc.at[dst_v.at[j - 1]], ssems[q]).wait()

                    @pl.when(j + PF - 1 < NCHA)
                    def _():
                        pltpu.async_copy(h_hbm.at[src_v.at[j + PF - 1]],
                                         bufs[q], sems[q])
                    pltpu.make_async_copy(h_hbm.at[src_v.at[j]],
                                          bufs[p], sems[p]).wait()
                    pltpu.async_copy(bufs[p], acc.at[dst_v.at[j]],
                                     ssems[p], add=True)
            return carry

        lax.fori_loop(0, NCHA, body, 0)
        pltpu.make_async_copy(bufs[(NCHA - 1) % PF],
                              acc.at[dst_v.at[NCHA - 1]],
                              ssems[(NCHA - 1) % PF]).wait()
        plsc.subcore_barrier()
        pltpu.sync_copy(acc.at[pl.ds(s * ZR, ZR)],
                        out_hbm.at[c, pl.ds(s * ZR, ZR)])

    return _agg


_BR = 256


def _norm_col(dpair, lo):
    d = dpair[0] + dpair[1]
    return lax.rsqrt(jnp.maximum(d[:, :1], lo))


def _tc1_body(x_ref, degs_ref, w1_ref, out_ref):
    ns = _norm_col(degs_ref[...], 1.0)
    out_ref[...] = jnp.dot(x_ref[...] * ns, w1_ref[...],
                           preferred_element_type=jnp.float32)


def _tc2_body(a_ref, degs_ref, degt_ref, w2_ref, b1_ref, out_ref):
    nd = _norm_col(degt_ref[...], 1.0)
    ns = _norm_col(degs_ref[...], 1.0)
    a = jnp.concatenate([a_ref[0], a_ref[1]], axis=1)
    h = jnp.maximum(a * nd + b1_ref[...], 0.0)
    out_ref[...] = jnp.dot(h * ns, w2_ref[...],
                           preferred_element_type=jnp.float32)


def _tc3_body(q_ref, degt_ref, b2_ref, out_ref):
    nd = _norm_col(degt_ref[...], 1.0)
    q = jnp.concatenate([q_ref[0], q_ref[1]], axis=1)
    out_ref[...] = q * nd + b2_ref[...]


def _row_spec(d):
    return pl.BlockSpec((_BR, d), lambda i: (i, 0))


def _deg_spec():
    return pl.BlockSpec((2, _BR, 16), lambda i: (0, i, 0))


def _full_spec(shape):
    return pl.BlockSpec(shape, lambda i: tuple(0 for _ in shape))


def kernel(x, edge_index, W1, b1, W2, b2):
    src = edge_index[0]
    dst = edge_index[1]
    pad = jnp.full((EP - EE,), PAD_IDX, jnp.int32)
    src_p = jnp.concatenate([src, pad])
    dst_p = jnp.concatenate([dst, pad])
    src_d = src_p.reshape(NW, NCHD, CHUNK)
    dst_d = dst_p.reshape(NW, NCHD, CHUNK)
    src2 = jnp.stack([2 * src_p, 2 * src_p + 1]).reshape(NC, NS, NCHA, CHUNK)
    dst_a = dst_p.reshape(NS, NCHA, CHUNK)
    ones16 = jnp.ones((CHUNK, 16), jnp.float32)
    z16 = jnp.zeros((ZR, 16), jnp.float32)
    z64 = jnp.zeros((ZR, D_H // 2), jnp.float32)
    z32 = jnp.zeros((ZR, N_CLS // 2), jnp.float32)

    deg_s, deg_t = _deg_kernel(src_d, dst_d, ones16, z16)

    grid = (NP // _BR,)
    h1 = pl.pallas_call(
        _tc1_body,
        grid=grid,
        in_specs=[_row_spec(D_IN), _deg_spec(), _full_spec((D_IN, D_H))],
        out_specs=_row_spec(D_H),
        out_shape=jax.ShapeDtypeStruct((NP, D_H), jnp.float32),
    )(x, deg_s, W1)

    agg1 = _make_agg_kernel(D_H)(
        h1.reshape(2 * NP, D_H // 2), src2, dst_a, z64)

    g2 = pl.pallas_call(
        _tc2_body,
        grid=grid,
        in_specs=[pl.BlockSpec((2, _BR, D_H // 2), lambda i: (0, i, 0)),
                  _deg_spec(), _deg_spec(),
                  _full_spec((D_H, N_CLS)), _full_spec((1, D_H))],
        out_specs=_row_spec(N_CLS),
        out_shape=jax.ShapeDtypeStruct((NP, N_CLS), jnp.float32),
    )(agg1, deg_s, deg_t, W2, b1.reshape(1, D_H))

    agg2 = _make_agg_kernel(N_CLS)(
        g2.reshape(2 * NP, N_CLS // 2), src2, dst_a, z32)

    out = pl.pallas_call(
        _tc3_body,
        grid=grid,
        in_specs=[pl.BlockSpec((2, _BR, N_CLS // 2), lambda i: (0, i, 0)),
                  _deg_spec(), _full_spec((1, N_CLS))],
        out_specs=_row_spec(N_CLS),
        out_shape=jax.ShapeDtypeStruct((NP, N_CLS), jnp.float32),
    )(agg2, deg_t, b2.reshape(1, N_CLS))

    return out[:NN]


# speedup vs baseline: 1.5488x; 1.0004x over previous
import functools

import jax
import jax.numpy as jnp
from jax import lax
from jax.experimental import pallas as pl
from jax.experimental.pallas import tpu as pltpu
from jax.experimental.pallas import tpu_sc as plsc

NN = 10000
EE = 320000
D_IN = 128
D_H = 128
N_CLS = 64

NC = 2
NS = 16
NW = NC * NS
CHUNK = 128
NCHD = 79
NCHA = 158
EP = NW * NCHD * CHUNK
PAD_IDX = 10008
NP = 10240
ZR = NP // NS
NB = 4

_mesh = plsc.VectorSubcoreMesh(core_axis_name="c", subcore_axis_name="s")
_sc_params = pltpu.CompilerParams(use_tc_tiling_on_sc=False)


@functools.partial(
    pl.kernel,
    out_type=[
        jax.ShapeDtypeStruct((NC, NP, 16), jnp.float32),
        jax.ShapeDtypeStruct((NC, NP, 16), jnp.float32),
    ],
    mesh=_mesh,
    scratch_types=[
        pltpu.VMEM((NCHD, CHUNK), jnp.int32),
        pltpu.VMEM((NCHD, CHUNK), jnp.int32),
        pltpu.VMEM((CHUNK, 16), jnp.float32),
        pltpu.VMEM_SHARED((NP, 16), jnp.float32),
        pltpu.VMEM_SHARED((NP, 16), jnp.float32),
        pltpu.SemaphoreType.DMA,
        pltpu.SemaphoreType.DMA,
    ],
    compiler_params=_sc_params,
)
def _deg_kernel(src_hbm, dst_hbm, ones_hbm, zeros_hbm, out_s, out_t,
                src_v, dst_v, ones_v, acc_s, acc_t, q0, q1):
    qs = [q0, q1]
    c = lax.axis_index("c")
    s = lax.axis_index("s")
    w = s * NC + c
    pltpu.sync_copy(zeros_hbm, acc_s.at[pl.ds(s * ZR, ZR)])
    pltpu.sync_copy(zeros_hbm, acc_t.at[pl.ds(s * ZR, ZR)])
    pltpu.sync_copy(src_hbm.at[w], src_v)
    pltpu.sync_copy(dst_hbm.at[w], dst_v)
    pltpu.sync_copy(ones_hbm, ones_v)
    plsc.subcore_barrier()

    def body(j, carry):
        for p in range(2):
            @pl.when(j % 2 == p)
            def _(p=p):
                @pl.when(j >= 2)
                def _():
                    pltpu.make_async_copy(
                        ones_v, acc_s.at[src_v.at[j - 2]], qs[p]).wait()
                    pltpu.make_async_copy(
                        ones_v, acc_t.at[dst_v.at[j - 2]], qs[p]).wait()
                pltpu.async_copy(ones_v, acc_s.at[src_v.at[j]], qs[p],
                                 add=True)
                pltpu.async_copy(ones_v, acc_t.at[dst_v.at[j]], qs[p],
                                 add=True)
        return carry

    lax.fori_loop(0, NCHD, body, 0)
    for j in (NCHD - 2, NCHD - 1):
        pltpu.make_async_copy(ones_v, acc_s.at[src_v.at[j]], qs[j % 2]).wait()
        pltpu.make_async_copy(ones_v, acc_t.at[dst_v.at[j]], qs[j % 2]).wait()
    plsc.subcore_barrier()
    pltpu.sync_copy(acc_s.at[pl.ds(s * ZR, ZR)], out_s.at[c, pl.ds(s * ZR, ZR)])
    pltpu.sync_copy(acc_t.at[pl.ds(s * ZR, ZR)], out_t.at[c, pl.ds(s * ZR, ZR)])


PF = 6


def _make_agg_kernel(D):
    HD = D // 2

    @functools.partial(
        pl.kernel,
        out_type=jax.ShapeDtypeStruct((NC, NP, HD), jnp.float32),
        mesh=_mesh,
        scratch_types=[
            pltpu.VMEM((NCHA, CHUNK), jnp.int32),
            pltpu.VMEM((NCHA, CHUNK), jnp.int32),
        ] + [pltpu.VMEM((CHUNK, HD), jnp.float32)] * PF + [
            pltpu.VMEM_SHARED((NP, HD), jnp.float32),
        ] + [pltpu.SemaphoreType.DMA] * (2 * PF),
        compiler_params=_sc_params,
    )
    def _agg(h_hbm, src_hbm, dst_hbm, zeros_hbm, out_hbm,
             src_v, dst_v, *rest):
        bufs = list(rest[:PF])
        acc = rest[PF]
        sems = list(rest[PF + 1:PF + 1 + PF])
        ssems = list(rest[PF + 1 + PF:])
        c = lax.axis_index("c")
        s = lax.axis_index("s")
        pltpu.sync_copy(zeros_hbm, acc.at[pl.ds(s * ZR, ZR)])
        pltpu.sync_copy(src_hbm.at[c, s], src_v)
        pltpu.sync_copy(dst_hbm.at[s], dst_v)
        plsc.subcore_barrier()

        for p in range(PF - 1):
            pltpu.async_copy(h_hbm.at[src_v.at[p]], bufs[p], sems[p])

        def body(j, carry):
            for p in range(PF):
                @pl.when(j % PF == p)
                def _(p=p):
                    q = (p + PF - 1) % PF

                    @pl.when(j > 0)
                    def _():
                        pltpu.make_async_copy(
                            bufs[q], acc.at[dst_v.at[j - 1]], ssems[q]).wait()

                    @pl.when(j + PF - 1 < NCHA)
                    def _():
                        pltpu.async_copy(h_hbm.at[src_v.at[j + PF - 1]],
                                         bufs[q], sems[q])
                    pltpu.make_async_copy(h_hbm.at[src_v.at[j]],
                                          bufs[p], sems[p]).wait()
                    pltpu.async_copy(bufs[p], acc.at[dst_v.at[j]],
                                     ssems[p], add=True)
            return carry

        lax.fori_loop(0, NCHA, body, 0)
        pltpu.make_async_copy(bufs[(NCHA - 1) % PF],
                              acc.at[dst_v.at[NCHA - 1]],
                              ssems[(NCHA - 1) % PF]).wait()
        plsc.subcore_barrier()
        pltpu.sync_copy(acc.at[pl.ds(s * ZR, ZR)],
                        out_hbm.at[c, pl.ds(s * ZR, ZR)])

    return _agg


_BR = 256


def _norm_col(dpair, lo):
    d = dpair[0] + dpair[1]
    return lax.rsqrt(jnp.maximum(d[:, :1], lo))


def _tc1_body(x_ref, degs_ref, w1_ref, out_ref):
    ns = _norm_col(degs_ref[...], 1.0)
    out_ref[...] = jnp.dot(x_ref[...] * ns, w1_ref[...],
                           preferred_element_type=jnp.float32)


def _tc2_body(a_ref, degs_ref, degt_ref, w2_ref, b1_ref, out_ref):
    nd = _norm_col(degt_ref[...], 1.0)
    ns = _norm_col(degs_ref[...], 1.0)
    a = jnp.concatenate([a_ref[0], a_ref[1]], axis=1)
    h = jnp.maximum(a * nd + b1_ref[...], 0.0)
    out_ref[...] = jnp.dot(h * ns, w2_ref[...],
                           preferred_element_type=jnp.float32)


def _tc3_body(q_ref, degt_ref, b2_ref, out_ref):
    nd = _norm_col(degt_ref[...], 1.0)
    q = jnp.concatenate([q_ref[0], q_ref[1]], axis=1)
    out_ref[...] = q * nd + b2_ref[...]


def _row_spec(d):
    return pl.BlockSpec((_BR, d), lambda i: (i, 0))


def _deg_spec():
    return pl.BlockSpec((2, _BR, 16), lambda i: (0, i, 0))


def _full_spec(shape):
    return pl.BlockSpec(shape, lambda i: tuple(0 for _ in shape))


def kernel(x, edge_index, W1, b1, W2, b2):
    src = edge_index[0]
    dst = edge_index[1]
    pad = jnp.full((EP - EE,), PAD_IDX, jnp.int32)
    src_p = jnp.concatenate([src, pad])
    dst_p = jnp.concatenate([dst, pad])
    src_d = src_p.reshape(NW, NCHD, CHUNK)
    dst_d = dst_p.reshape(NW, NCHD, CHUNK)
    src2 = jnp.stack([2 * src_p, 2 * src_p + 1]).reshape(NC, NS, NCHA, CHUNK)
    dst_a = dst_p.reshape(NS, NCHA, CHUNK)
    ones16 = jnp.ones((CHUNK, 16), jnp.float32)
    z16 = jnp.zeros((ZR, 16), jnp.float32)
    z64 = jnp.zeros((ZR, D_H // 2), jnp.float32)
    z32 = jnp.zeros((ZR, N_CLS // 2), jnp.float32)

    deg_s, deg_t = _deg_kernel(src_d, dst_d, ones16, z16)

    grid = (NP // _BR,)
    h1 = pl.pallas_call(
        _tc1_body,
        grid=grid,
        in_specs=[_row_spec(D_IN), _deg_spec(), _full_spec((D_IN, D_H))],
        out_specs=_row_spec(D_H),
        out_shape=jax.ShapeDtypeStruct((NP, D_H), jnp.float32),
    )(x, deg_s, W1)

    agg1 = _make_agg_kernel(D_H)(
        h1.reshape(2 * NP, D_H // 2), src2, dst_a, z64)

    g2 = pl.pallas_call(
        _tc2_body,
        grid=grid,
        in_specs=[pl.BlockSpec((2, _BR, D_H // 2), lambda i: (0, i, 0)),
                  _deg_spec(), _deg_spec(),
                  _full_spec((D_H, N_CLS)), _full_spec((1, D_H))],
        out_specs=_row_spec(N_CLS),
        out_shape=jax.ShapeDtypeStruct((NP, N_CLS), jnp.float32),
    )(agg1, deg_s, deg_t, W2, b1.reshape(1, D_H))

    agg2 = _make_agg_kernel(N_CLS)(
        g2.reshape(2 * NP, N_CLS // 2), src2, dst_a, z32)

    out = pl.pallas_call(
        _tc3_body,
        grid=grid,
        in_specs=[pl.BlockSpec((2, _BR, N_CLS // 2), lambda i: (0, i, 0)),
                  _deg_spec(), _full_spec((1, N_CLS))],
        out_specs=_row_spec(N_CLS),
        out_shape=jax.ShapeDtypeStruct((NP, N_CLS), jnp.float32),
    )(agg2, deg_t, b2.reshape(1, N_CLS))

    return out[:NN]
